# baseline (speedup 1.0000x reference)
import functools

import jax
import jax.numpy as jnp
from jax import lax
from jax.experimental import pallas as pl
from jax.experimental.pallas import tpu as pltpu
from jax.experimental.pallas import tpu_sc as plsc

D = 128
W = D // 2
L = 16
NC = 2
NS = 16
NW = NC * NS
NBUF = 4


def _dot_body(hp_hbm, cidx_hbm, out_hbm,
              h_sp, cidx, rows, scores2, tr, gsems, osems,
              *, epw, chunk, n_nodes):
    sid = lax.axis_index("s")
    wid = sid * NC + lax.axis_index("c")
    base = wid * epw
    nchunks = epw // chunk
    ngroups = chunk // L
    cw = 2 * chunk

    rows_per_tile = (n_nodes // NS) // 8 * 8
    tail = n_nodes - rows_per_tile * NS
    pltpu.sync_copy(hp_hbm.at[pl.ds(sid * rows_per_tile, rows_per_tile)],
                    h_sp.at[pl.ds(sid * rows_per_tile, rows_per_tile)])
    if tail:
        @pl.when(sid == NS - 1)
        def _():
            pltpu.sync_copy(hp_hbm.at[pl.ds(rows_per_tile * NS, tail)],
                            h_sp.at[pl.ds(rows_per_tile * NS, tail)])
    pltpu.sync_copy(cidx_hbm.at[pl.ds(wid * 2 * epw, 2 * epw)], cidx)
    plsc.subcore_barrier()

    def issue(c, slot):
        ic = cidx.at[pl.ds(c * cw, cw)]
        pltpu.async_copy(h_sp.at[ic], rows.at[slot], gsems.at[slot])

    def wait_gather(c, slot):
        ic = cidx.at[pl.ds(c * cw, cw)]
        pltpu.make_async_copy(h_sp.at[ic], rows.at[slot],
                              gsems.at[slot]).wait()

    col16 = lax.iota(jnp.int32, L) * (L + 1)

    def compute(slot, sslot):
        rr = rows.at[slot]
        sc = scores2.at[sslot]

        def group_body(g, gcarry):
            @plsc.parallel_loop(0, L, step=1, unroll=4)
            def edge_body(e):
                ea = g * L + e
                acc_lo = None
                acc_hi = None
                for k in range(W // L):
                    uw = rr[ea, pl.ds(k * L, L)]
                    vw = rr[chunk + ea, pl.ds(k * L, L)]
                    ub = plsc.bitcast(uw, jnp.bfloat16)
                    vb = plsc.bitcast(vw, jnp.bfloat16)
                    prod = ub * vb
                    pe, po = plsc.unpack(prod,
                                         format=plsc.PackFormat.INTERLEAVED)
                    acc_lo = pe if acc_lo is None else acc_lo + pe
                    acc_hi = po if acc_hi is None else acc_hi + po
                acc = acc_lo + acc_hi
                plsc.store_scatter(tr, [col16 + e], acc)

            terms = [tr[pl.ds(i * (L + 1), L)] for i in range(L)]
            while len(terms) > 1:
                terms = [a + b for a, b in zip(terms[::2], terms[1::2])]
            sc[pl.ds(g * L, L)] = terms[0]
            return gcarry

        lax.fori_loop(0, ngroups, group_body, 0)

    def store_scores(c, sslot):
        pltpu.async_copy(scores2.at[sslot],
                         out_hbm.at[pl.ds(base + c * chunk, chunk)],
                         osems.at[sslot])

    def wait_store(c, sslot):
        pltpu.make_async_copy(scores2.at[sslot],
                              out_hbm.at[pl.ds(base + c * chunk, chunk)],
                              osems.at[sslot]).wait()

    for s in range(min(NBUF - 1, nchunks)):
        issue(s, s)

    def chunk_body(c, carry):
        def do(slot):
            wait_gather(c, slot)

            @pl.when(c + NBUF - 1 < nchunks)
            def _():
                issue(c + NBUF - 1, (slot + NBUF - 1) % NBUF)

            sslot = slot % 2

            @pl.when(c >= 2)
            def _():
                wait_store(c - 2, sslot)

            compute(slot, sslot)
            store_scores(c, sslot)

        for s in range(NBUF):
            @pl.when(c % NBUF == s)
            def _(s=s):
                do(s)

        return carry

    lax.fori_loop(0, nchunks, chunk_body, 0)
    wait_store(nchunks - 2, (nchunks - 2) % 2)
    wait_store(nchunks - 1, (nchunks - 1) % 2)


def kernel(h, edge_index):
    E = edge_index.shape[1]
    epw = E // NW
    chunk = 80
    nchunks = epw // chunk
    n_nodes = h.shape[0]
    hb = h.astype(jnp.bfloat16)
    hp = jax.lax.bitcast_convert_type(
        hb.reshape(n_nodes, W, 2), jnp.int32)
    src = edge_index[0].astype(jnp.int32).reshape(NW, nchunks, chunk)
    dst = edge_index[1].astype(jnp.int32).reshape(NW, nchunks, chunk)
    cidx = jnp.concatenate([src, dst], axis=-1).reshape(-1)
    mesh = plsc.VectorSubcoreMesh(core_axis_name="c", subcore_axis_name="s")
    body = functools.partial(_dot_body, epw=epw, chunk=chunk,
                             n_nodes=n_nodes)
    f = pl.kernel(
        body,
        mesh=mesh,
        compiler_params=pltpu.CompilerParams(needs_layout_passes=False,
                                             use_tc_tiling_on_sc=False),
        out_type=jax.ShapeDtypeStruct((E,), jnp.float32),
        scratch_types=[
            pltpu.VMEM_SHARED((n_nodes, W), jnp.int32),
            pltpu.VMEM((2 * epw,), jnp.int32),
            pltpu.VMEM((NBUF, 2 * chunk, W), jnp.int32),
            pltpu.VMEM((2, chunk), jnp.float32),
            pltpu.VMEM((L * (L + 1),), jnp.float32),
            pltpu.SemaphoreType.DMA((NBUF,)),
            pltpu.SemaphoreType.DMA((2,)),
        ],
    )
    return f(hp, cidx)


# speedup vs baseline: 1.0206x; 1.0206x over previous
import functools

import jax
import jax.numpy as jnp
from jax import lax
from jax.experimental import pallas as pl
from jax.experimental.pallas import tpu as pltpu
from jax.experimental.pallas import tpu_sc as plsc

D = 128
W = D // 2
L = 16
NC = 2
NS = 16
NW = NC * NS
CHUNK = 320
TAIL = 80


def _dot_body(hp_hbm, cidx_hbm, out_hbm,
              h_sp, cidx_v, rows, scores2, tr, gsems, isems, osems,
              *, epw, n_nodes):
    sid = lax.axis_index("s")
    wid = sid * NC + lax.axis_index("c")
    base = wid * epw
    nfull = (epw - TAIL) // CHUNK
    tpw = 2 * epw
    cw = 2 * CHUNK
    tw = 2 * TAIL
    ibase = wid * tpw
    tail_off = nfull * cw

    rows_per_tile = (n_nodes // NS) // 8 * 8
    rtail = n_nodes - rows_per_tile * NS
    pltpu.sync_copy(hp_hbm.at[pl.ds(sid * rows_per_tile, rows_per_tile)],
                    h_sp.at[pl.ds(sid * rows_per_tile, rows_per_tile)])
    if rtail:
        @pl.when(sid == NS - 1)
        def _():
            pltpu.sync_copy(hp_hbm.at[pl.ds(rows_per_tile * NS, rtail)],
                            h_sp.at[pl.ds(rows_per_tile * NS, rtail)])
    pltpu.sync_copy(cidx_hbm.at[pl.ds(ibase, cw)], cidx_v.at[0])
    pltpu.async_copy(cidx_hbm.at[pl.ds(ibase + cw, cw)], cidx_v.at[1],
                     isems.at[1])
    plsc.subcore_barrier()

    def issue_gather(slot):
        pltpu.async_copy(h_sp.at[cidx_v.at[slot]], rows.at[slot],
                         gsems.at[slot])

    def wait_gather(slot):
        pltpu.make_async_copy(h_sp.at[cidx_v.at[slot]], rows.at[slot],
                              gsems.at[slot]).wait()

    def prefetch_idx(c, slot):
        pltpu.async_copy(cidx_hbm.at[pl.ds(ibase + c * cw, cw)],
                         cidx_v.at[slot], isems.at[slot])

    def wait_idx(slot):
        pltpu.make_async_copy(cidx_hbm.at[pl.ds(ibase, cw)],
                              cidx_v.at[slot], isems.at[slot]).wait()

    col16 = lax.iota(jnp.int32, L) * (L + 1)

    def compute(slot, sslot, ngroups, voff):
        rr = rows.at[slot]
        sc = scores2.at[sslot]

        def group_body(g, gcarry):
            @plsc.parallel_loop(0, L, step=1, unroll=4)
            def edge_body(e):
                ea = g * L + e
                acc_lo = None
                acc_hi = None
                for k in range(W // L):
                    uw = rr[ea, pl.ds(k * L, L)]
                    vw = rr[voff + ea, pl.ds(k * L, L)]
                    ub = plsc.bitcast(uw, jnp.bfloat16)
                    vb = plsc.bitcast(vw, jnp.bfloat16)
                    prod = ub * vb
                    pe, po = plsc.unpack(prod,
                                         format=plsc.PackFormat.INTERLEAVED)
                    acc_lo = pe if acc_lo is None else acc_lo + pe
                    acc_hi = po if acc_hi is None else acc_hi + po
                acc = acc_lo + acc_hi
                plsc.store_scatter(tr, [col16 + e], acc)

            terms = [tr[pl.ds(i * (L + 1), L)] for i in range(L)]
            while len(terms) > 1:
                terms = [a + b for a, b in zip(terms[::2], terms[1::2])]
            sc[pl.ds(g * L, L)] = terms[0]
            return gcarry

        lax.fori_loop(0, ngroups, group_body, 0)

    def store_scores(c, sslot):
        pltpu.async_copy(scores2.at[sslot],
                         out_hbm.at[pl.ds(base + c * CHUNK, CHUNK)],
                         osems.at[sslot])

    def wait_store(c, sslot):
        pltpu.make_async_copy(scores2.at[sslot],
                              out_hbm.at[pl.ds(base + c * CHUNK, CHUNK)],
                              osems.at[sslot]).wait()

    issue_gather(0)

    def chunk_body(c, carry):
        def do(s, o):
            wait_gather(s)

            @pl.when(c + 1 < nfull)
            def _():
                wait_idx(o)
                issue_gather(o)

            @pl.when(c + 2 < nfull)
            def _():
                prefetch_idx(c + 2, s)

            @pl.when(c + 2 == nfull)
            def _():
                pltpu.async_copy(
                    cidx_hbm.at[pl.ds(ibase + tail_off, tw)],
                    cidx_v.at[s, pl.ds(0, tw)], isems.at[s])

            @pl.when(c >= 2)
            def _():
                wait_store(c - 2, s)

            compute(s, s, CHUNK // L, CHUNK)
            store_scores(c, s)

        @pl.when(c % 2 == 0)
        def _():
            do(0, 1)

        @pl.when(c % 2 == 1)
        def _():
            do(1, 0)

        return carry

    lax.fori_loop(0, nfull, chunk_body, 0)

    tslot = (nfull - 2) % 2
    pltpu.make_async_copy(cidx_hbm.at[pl.ds(ibase + tail_off, tw)],
                          cidx_v.at[tslot, pl.ds(0, tw)],
                          isems.at[tslot]).wait()
    tidx = cidx_v.at[tslot, pl.ds(0, tw)]
    trows = rows.at[tslot, pl.ds(0, tw)]
    pltpu.async_copy(h_sp.at[tidx], trows, gsems.at[tslot])
    pltpu.make_async_copy(h_sp.at[tidx], trows, gsems.at[tslot]).wait()
    wait_store(nfull - 2, (nfull - 2) % 2)
    wait_store(nfull - 1, (nfull - 1) % 2)
    compute(tslot, tslot, TAIL // L, TAIL)
    tsc = scores2.at[tslot, pl.ds(0, TAIL)]
    pltpu.async_copy(tsc, out_hbm.at[pl.ds(base + nfull * CHUNK, TAIL)],
                     osems.at[tslot])
    pltpu.make_async_copy(tsc,
                          out_hbm.at[pl.ds(base + nfull * CHUNK, TAIL)],
                          osems.at[tslot]).wait()


def kernel(h, edge_index):
    E = edge_index.shape[1]
    epw = E // NW
    n_nodes = h.shape[0]
    nfull = (epw - TAIL) // CHUNK
    hb = h.astype(jnp.bfloat16)
    hp = jax.lax.bitcast_convert_type(
        hb.reshape(n_nodes, W, 2), jnp.int32)
    src = edge_index[0].astype(jnp.int32).reshape(NW, epw)
    dst = edge_index[1].astype(jnp.int32).reshape(NW, epw)
    nf = nfull * CHUNK
    fullb = jnp.concatenate(
        [src[:, :nf].reshape(NW, nfull, CHUNK),
         dst[:, :nf].reshape(NW, nfull, CHUNK)],
        axis=-1).reshape(NW, nfull * 2 * CHUNK)
    tailb = jnp.concatenate([src[:, nf:], dst[:, nf:]], axis=-1)
    cidx = jnp.concatenate([fullb, tailb], axis=-1).reshape(-1)
    mesh = plsc.VectorSubcoreMesh(core_axis_name="c", subcore_axis_name="s")
    body = functools.partial(_dot_body, epw=epw, n_nodes=n_nodes)
    f = pl.kernel(
        body,
        mesh=mesh,
        compiler_params=pltpu.CompilerParams(needs_layout_passes=False,
                                             use_tc_tiling_on_sc=False),
        out_type=jax.ShapeDtypeStruct((E,), jnp.float32),
        scratch_types=[
            pltpu.VMEM_SHARED((n_nodes, W), jnp.int32),
            pltpu.VMEM((2, 2 * CHUNK), jnp.int32),
            pltpu.VMEM((2, 2 * CHUNK, W), jnp.int32),
            pltpu.VMEM((2, CHUNK), jnp.float32),
            pltpu.VMEM((L * (L + 1),), jnp.float32),
            pltpu.SemaphoreType.DMA((2,)),
            pltpu.SemaphoreType.DMA((2,)),
            pltpu.SemaphoreType.DMA((2,)),
        ],
    )
    return f(hp, cidx)


# speedup vs baseline: 1.0760x; 1.0542x over previous
import functools

import jax
import jax.numpy as jnp
from jax import lax
from jax.experimental import pallas as pl
from jax.experimental.pallas import tpu as pltpu
from jax.experimental.pallas import tpu_sc as plsc

D = 128
W = D // 2
L = 16
NC = 2
NS = 16
NW = NC * NS
CHUNK = 320
TAIL = 80


def _dot_body(hp_hbm, cidx_hbm, out_hbm,
              h_sp, cidx_v, rows, scores2, tr, gsems, isems, osems,
              *, epw, n_nodes):
    sid = lax.axis_index("s")
    wid = sid * NC + lax.axis_index("c")
    base = wid * epw
    nfull = (epw - TAIL) // CHUNK
    tpw = 2 * epw
    cw = 2 * CHUNK
    tw = 2 * TAIL
    ibase = wid * tpw
    tail_off = nfull * cw

    rows_per_tile = (n_nodes // NS) // 8 * 8
    rtail = n_nodes - rows_per_tile * NS
    pltpu.sync_copy(hp_hbm.at[pl.ds(sid * rows_per_tile, rows_per_tile)],
                    h_sp.at[pl.ds(sid * rows_per_tile, rows_per_tile)])
    if rtail:
        @pl.when(sid == NS - 1)
        def _():
            pltpu.sync_copy(hp_hbm.at[pl.ds(rows_per_tile * NS, rtail)],
                            h_sp.at[pl.ds(rows_per_tile * NS, rtail)])
    pltpu.sync_copy(cidx_hbm.at[pl.ds(ibase, cw)], cidx_v.at[0])
    pltpu.async_copy(cidx_hbm.at[pl.ds(ibase + cw, cw)], cidx_v.at[1],
                     isems.at[1])
    plsc.subcore_barrier()

    def issue_gather(slot):
        pltpu.async_copy(h_sp.at[cidx_v.at[slot]], rows.at[slot],
                         gsems.at[slot])

    def wait_gather(slot):
        pltpu.make_async_copy(h_sp.at[cidx_v.at[slot]], rows.at[slot],
                              gsems.at[slot]).wait()

    def prefetch_idx(c, slot):
        pltpu.async_copy(cidx_hbm.at[pl.ds(ibase + c * cw, cw)],
                         cidx_v.at[slot], isems.at[slot])

    def wait_idx(slot):
        pltpu.make_async_copy(cidx_hbm.at[pl.ds(ibase, cw)],
                              cidx_v.at[slot], isems.at[slot]).wait()

    lane = lax.iota(jnp.int32, L)

    def compute(slot, sslot, ngroups, voff):
        rr = rows.at[slot]
        sc = scores2.at[sslot]

        def group_body(g, gcarry):
            @plsc.parallel_loop(0, L, step=1, unroll=4,
                                carry=jnp.zeros((L,), jnp.float32))
            def edge_body(e, res):
                ea = g * L + e
                acc_lo = None
                acc_hi = None
                for k in range(W // L):
                    uw = rr[ea, pl.ds(k * L, L)]
                    vw = rr[voff + ea, pl.ds(k * L, L)]
                    ub = plsc.bitcast(uw, jnp.bfloat16)
                    vb = plsc.bitcast(vw, jnp.bfloat16)
                    prod = ub * vb
                    pe, po = plsc.unpack(prod,
                                         format=plsc.PackFormat.INTERLEAVED)
                    acc_lo = pe if acc_lo is None else acc_lo + pe
                    acc_hi = po if acc_hi is None else acc_hi + po
                acc = acc_lo + acc_hi
                s = jnp.sum(acc)
                return jnp.where(lane == e, jnp.full((L,), s, jnp.float32),
                                 res)

            sc[pl.ds(g * L, L)] = edge_body
            return gcarry

        lax.fori_loop(0, ngroups, group_body, 0)

    def store_scores(c, sslot):
        pltpu.async_copy(scores2.at[sslot],
                         out_hbm.at[pl.ds(base + c * CHUNK, CHUNK)],
                         osems.at[sslot])

    def wait_store(c, sslot):
        pltpu.make_async_copy(scores2.at[sslot],
                              out_hbm.at[pl.ds(base + c * CHUNK, CHUNK)],
                              osems.at[sslot]).wait()

    issue_gather(0)

    def chunk_body(c, carry):
        def do(s, o):
            wait_gather(s)

            @pl.when(c + 1 < nfull)
            def _():
                wait_idx(o)
                issue_gather(o)

            @pl.when(c + 2 < nfull)
            def _():
                prefetch_idx(c + 2, s)

            @pl.when(c + 2 == nfull)
            def _():
                pltpu.async_copy(
                    cidx_hbm.at[pl.ds(ibase + tail_off, tw)],
                    cidx_v.at[s, pl.ds(0, tw)], isems.at[s])

            @pl.when(c >= 2)
            def _():
                wait_store(c - 2, s)

            compute(s, s, CHUNK // L, CHUNK)
            store_scores(c, s)

        @pl.when(c % 2 == 0)
        def _():
            do(0, 1)

        @pl.when(c % 2 == 1)
        def _():
            do(1, 0)

        return carry

    lax.fori_loop(0, nfull, chunk_body, 0)

    tslot = (nfull - 2) % 2
    pltpu.make_async_copy(cidx_hbm.at[pl.ds(ibase + tail_off, tw)],
                          cidx_v.at[tslot, pl.ds(0, tw)],
                          isems.at[tslot]).wait()
    tidx = cidx_v.at[tslot, pl.ds(0, tw)]
    trows = rows.at[tslot, pl.ds(0, tw)]
    pltpu.async_copy(h_sp.at[tidx], trows, gsems.at[tslot])
    pltpu.make_async_copy(h_sp.at[tidx], trows, gsems.at[tslot]).wait()
    wait_store(nfull - 2, (nfull - 2) % 2)
    wait_store(nfull - 1, (nfull - 1) % 2)
    compute(tslot, tslot, TAIL // L, TAIL)
    tsc = scores2.at[tslot, pl.ds(0, TAIL)]
    pltpu.async_copy(tsc, out_hbm.at[pl.ds(base + nfull * CHUNK, TAIL)],
                     osems.at[tslot])
    pltpu.make_async_copy(tsc,
                          out_hbm.at[pl.ds(base + nfull * CHUNK, TAIL)],
                          osems.at[tslot]).wait()


def kernel(h, edge_index):
    E = edge_index.shape[1]
    epw = E // NW
    n_nodes = h.shape[0]
    nfull = (epw - TAIL) // CHUNK
    hb = h.astype(jnp.bfloat16)
    hp = jax.lax.bitcast_convert_type(
        hb.reshape(n_nodes, W, 2), jnp.int32)
    src = edge_index[0].astype(jnp.int32).reshape(NW, epw)
    dst = edge_index[1].astype(jnp.int32).reshape(NW, epw)
    nf = nfull * CHUNK
    fullb = jnp.concatenate(
        [src[:, :nf].reshape(NW, nfull, CHUNK),
         dst[:, :nf].reshape(NW, nfull, CHUNK)],
        axis=-1).reshape(NW, nfull * 2 * CHUNK)
    tailb = jnp.concatenate([src[:, nf:], dst[:, nf:]], axis=-1)
    cidx = jnp.concatenate([fullb, tailb], axis=-1).reshape(-1)
    mesh = plsc.VectorSubcoreMesh(core_axis_name="c", subcore_axis_name="s")
    body = functools.partial(_dot_body, epw=epw, n_nodes=n_nodes)
    f = pl.kernel(
        body,
        mesh=mesh,
        compiler_params=pltpu.CompilerParams(needs_layout_passes=False,
                                             use_tc_tiling_on_sc=False),
        out_type=jax.ShapeDtypeStruct((E,), jnp.float32),
        scratch_types=[
            pltpu.VMEM_SHARED((n_nodes, W), jnp.int32),
            pltpu.VMEM((2, 2 * CHUNK), jnp.int32),
            pltpu.VMEM((2, 2 * CHUNK, W), jnp.int32),
            pltpu.VMEM((2, CHUNK), jnp.float32),
            pltpu.VMEM((L * (L + 1),), jnp.float32),
            pltpu.SemaphoreType.DMA((2,)),
            pltpu.SemaphoreType.DMA((2,)),
            pltpu.SemaphoreType.DMA((2,)),
        ],
    )
    return f(hp, cidx)


# speedup vs baseline: 1.0881x; 1.0113x over previous
import functools

import jax
import jax.numpy as jnp
from jax import lax
from jax.experimental import pallas as pl
from jax.experimental.pallas import tpu as pltpu
from jax.experimental.pallas import tpu_sc as plsc

D = 128
W = D // 2
L = 16
NC = 2
NS = 16
NW = NC * NS
CHUNK = 320
TAIL = 80


def _dot_body(hp_hbm, cidx_hbm, out_hbm,
              h_sp, cidx_v, rows, scores2, tr, gsems, isems, osems,
              *, epw, n_nodes):
    sid = lax.axis_index("s")
    wid = sid * NC + lax.axis_index("c")
    base = wid * epw
    nfull = (epw - TAIL) // CHUNK
    tpw = 2 * epw
    cw = 2 * CHUNK
    tw = 2 * TAIL
    ibase = wid * tpw
    tail_off = nfull * cw

    rows_per_tile = (n_nodes // NS) // 8 * 8
    rtail = n_nodes - rows_per_tile * NS
    pltpu.sync_copy(hp_hbm.at[pl.ds(sid * rows_per_tile, rows_per_tile)],
                    h_sp.at[pl.ds(sid * rows_per_tile, rows_per_tile)])
    if rtail:
        @pl.when(sid == NS - 1)
        def _():
            pltpu.sync_copy(hp_hbm.at[pl.ds(rows_per_tile * NS, rtail)],
                            h_sp.at[pl.ds(rows_per_tile * NS, rtail)])
    pltpu.sync_copy(cidx_hbm.at[pl.ds(ibase, cw)], cidx_v.at[0])
    pltpu.async_copy(cidx_hbm.at[pl.ds(ibase + cw, cw)], cidx_v.at[1],
                     isems.at[1])
    plsc.subcore_barrier()

    def issue_gather(slot):
        pltpu.async_copy(h_sp.at[cidx_v.at[slot]], rows.at[slot],
                         gsems.at[slot])

    def wait_gather(slot):
        pltpu.make_async_copy(h_sp.at[cidx_v.at[slot]], rows.at[slot],
                              gsems.at[slot]).wait()

    def prefetch_idx(c, slot):
        pltpu.async_copy(cidx_hbm.at[pl.ds(ibase + c * cw, cw)],
                         cidx_v.at[slot], isems.at[slot])

    def wait_idx(slot):
        pltpu.make_async_copy(cidx_hbm.at[pl.ds(ibase, cw)],
                              cidx_v.at[slot], isems.at[slot]).wait()

    lane = lax.iota(jnp.int32, L)

    def compute(slot, sslot, ngroups, voff):
        rr = rows.at[slot]
        sc = scores2.at[sslot]

        def group_body(g, gcarry):
            @plsc.parallel_loop(0, L, step=1, unroll=4,
                                carry=jnp.zeros((L,), jnp.float32))
            def edge_body(e, res):
                ea = g * L + e
                pacc = None
                for k in range(W // L):
                    uw = rr[ea, pl.ds(k * L, L)]
                    vw = rr[voff + ea, pl.ds(k * L, L)]
                    ub = plsc.bitcast(uw, jnp.bfloat16)
                    vb = plsc.bitcast(vw, jnp.bfloat16)
                    prod = ub * vb
                    pacc = prod if pacc is None else pacc + prod
                pe, po = plsc.unpack(pacc,
                                     format=plsc.PackFormat.INTERLEAVED)
                acc = pe + po
                s = jnp.sum(acc)
                return jnp.where(lane == e, jnp.full((L,), s, jnp.float32),
                                 res)

            sc[pl.ds(g * L, L)] = edge_body
            return gcarry

        lax.fori_loop(0, ngroups, group_body, 0)

    def store_scores(c, sslot):
        pltpu.async_copy(scores2.at[sslot],
                         out_hbm.at[pl.ds(base + c * CHUNK, CHUNK)],
                         osems.at[sslot])

    def wait_store(c, sslot):
        pltpu.make_async_copy(scores2.at[sslot],
                              out_hbm.at[pl.ds(base + c * CHUNK, CHUNK)],
                              osems.at[sslot]).wait()

    issue_gather(0)

    def chunk_body(c, carry):
        def do(s, o):
            wait_gather(s)

            @pl.when(c + 1 < nfull)
            def _():
                wait_idx(o)
                issue_gather(o)

            @pl.when(c + 2 < nfull)
            def _():
                prefetch_idx(c + 2, s)

            @pl.when(c + 2 == nfull)
            def _():
                pltpu.async_copy(
                    cidx_hbm.at[pl.ds(ibase + tail_off, tw)],
                    cidx_v.at[s, pl.ds(0, tw)], isems.at[s])

            @pl.when(c >= 2)
            def _():
                wait_store(c - 2, s)

            compute(s, s, CHUNK // L, CHUNK)
            store_scores(c, s)

        @pl.when(c % 2 == 0)
        def _():
            do(0, 1)

        @pl.when(c % 2 == 1)
        def _():
            do(1, 0)

        return carry

    lax.fori_loop(0, nfull, chunk_body, 0)

    tslot = (nfull - 2) % 2
    pltpu.make_async_copy(cidx_hbm.at[pl.ds(ibase + tail_off, tw)],
                          cidx_v.at[tslot, pl.ds(0, tw)],
                          isems.at[tslot]).wait()
    tidx = cidx_v.at[tslot, pl.ds(0, tw)]
    trows = rows.at[tslot, pl.ds(0, tw)]
    pltpu.async_copy(h_sp.at[tidx], trows, gsems.at[tslot])
    pltpu.make_async_copy(h_sp.at[tidx], trows, gsems.at[tslot]).wait()
    wait_store(nfull - 2, (nfull - 2) % 2)
    wait_store(nfull - 1, (nfull - 1) % 2)
    compute(tslot, tslot, TAIL // L, TAIL)
    tsc = scores2.at[tslot, pl.ds(0, TAIL)]
    pltpu.async_copy(tsc, out_hbm.at[pl.ds(base + nfull * CHUNK, TAIL)],
                     osems.at[tslot])
    pltpu.make_async_copy(tsc,
                          out_hbm.at[pl.ds(base + nfull * CHUNK, TAIL)],
                          osems.at[tslot]).wait()


def kernel(h, edge_index):
    E = edge_index.shape[1]
    epw = E // NW
    n_nodes = h.shape[0]
    nfull = (epw - TAIL) // CHUNK
    hb = h.astype(jnp.bfloat16)
    hp = jax.lax.bitcast_convert_type(
        hb.reshape(n_nodes, W, 2), jnp.int32)
    src = edge_index[0].astype(jnp.int32).reshape(NW, epw)
    dst = edge_index[1].astype(jnp.int32).reshape(NW, epw)
    nf = nfull * CHUNK
    fullb = jnp.concatenate(
        [src[:, :nf].reshape(NW, nfull, CHUNK),
         dst[:, :nf].reshape(NW, nfull, CHUNK)],
        axis=-1).reshape(NW, nfull * 2 * CHUNK)
    tailb = jnp.concatenate([src[:, nf:], dst[:, nf:]], axis=-1)
    cidx = jnp.concatenate([fullb, tailb], axis=-1).reshape(-1)
    mesh = plsc.VectorSubcoreMesh(core_axis_name="c", subcore_axis_name="s")
    body = functools.partial(_dot_body, epw=epw, n_nodes=n_nodes)
    f = pl.kernel(
        body,
        mesh=mesh,
        compiler_params=pltpu.CompilerParams(needs_layout_passes=False,
                                             use_tc_tiling_on_sc=False),
        out_type=jax.ShapeDtypeStruct((E,), jnp.float32),
        scratch_types=[
            pltpu.VMEM_SHARED((n_nodes, W), jnp.int32),
            pltpu.VMEM((2, 2 * CHUNK), jnp.int32),
            pltpu.VMEM((2, 2 * CHUNK, W), jnp.int32),
            pltpu.VMEM((2, CHUNK), jnp.float32),
            pltpu.VMEM((L * (L + 1),), jnp.float32),
            pltpu.SemaphoreType.DMA((2,)),
            pltpu.SemaphoreType.DMA((2,)),
            pltpu.SemaphoreType.DMA((2,)),
        ],
    )
    return f(hp, cidx)
